# baseline (speedup 1.0000x reference)
import functools
import math

import jax
import jax.numpy as jnp
from jax import lax
from jax.experimental import pallas as pl
from jax.experimental.pallas import tpu as pltpu
from jax.experimental.pallas import tpu_sc as plsc

D_MODEL = 64
SCALE = math.sqrt(D_MODEL)
LANES = 16

_info = plsc.get_sparse_core_info()
NUM_CORES = _info.num_cores
NUM_SUBCORES = _info.num_subcores
NUM_WORKERS = NUM_CORES * NUM_SUBCORES

NBUF = 2
RBLK = 128
SUBL = 8
TILE_N = RBLK * SUBL


def _make_lookup(n_rows, row_len, d_model):
    assert n_rows == NUM_WORKERS * RBLK
    n_chunks = row_len
    n_jt = d_model // SUBL
    n_groups = n_chunks // NBUF
    assert n_chunks % NBUF == 0
    assert n_groups >= 4 and n_groups % 2 == 0
    n_pairs = (n_groups - 2) // 2

    mesh = plsc.VectorSubcoreMesh(core_axis_name="c", subcore_axis_name="s")

    @functools.partial(
        pl.kernel,
        mesh=mesh,
        compiler_params=pltpu.CompilerParams(use_tc_tiling_on_sc=False,
                                             needs_layout_passes=False),
        out_type=jax.ShapeDtypeStruct(
            (row_len, n_jt, NUM_WORKERS, SUBL, RBLK), jnp.float32),
        scratch_types=[
            pltpu.VMEM((row_len, RBLK), jnp.int32),
            pltpu.VMEM((2 * NBUF, RBLK, d_model), jnp.float32),
            pltpu.VMEM((2 * NBUF, n_jt, SUBL, RBLK), jnp.float32),
            pltpu.SemaphoreType.DMA((2 * NBUF,)),
            pltpu.SemaphoreType.DMA((2 * NBUF,)),
        ],
    )
    def lookup(x_hbm, table_hbm, out_hbm, idx_v, rows_v, stage_v, gsem, ssem):
        wid = lax.axis_index("s") * NUM_CORES + lax.axis_index("c")

        pltpu.sync_copy(x_hbm.at[:, pl.ds(wid * RBLK, RBLK)], idx_v)

        def gather_start(g, s):
            pltpu.async_copy(table_hbm.at[idx_v.at[g]], rows_v.at[s],
                             gsem.at[s])

        def gather_wait(g, s):
            pltpu.make_async_copy(table_hbm.at[idx_v.at[g]], rows_v.at[s],
                                  gsem.at[s]).wait()

        def scatter_start(g, s):
            for jt in range(n_jt):
                pltpu.async_copy(stage_v.at[s, jt],
                                 out_hbm.at[g, jt, wid],
                                 ssem.at[s])

        def scatter_wait(g, s):
            for jt in range(n_jt):
                pltpu.make_async_copy(stage_v.at[s, jt],
                                      out_hbm.at[g, jt, wid],
                                      ssem.at[s]).wait()

        iotas = [lax.iota(jnp.int32, LANES) + rb * LANES
                 for rb in range(RBLK // LANES)]

        def scale(s):
            def col_body(j, c):
                jt = j // SUBL
                jr = j % SUBL
                colv = jnp.full((LANES,), j, dtype=jnp.int32)
                for rb in range(RBLK // LANES):
                    v = plsc.load_gather(rows_v.at[s], [iotas[rb], colv])
                    stage_v[s, jt, jr, pl.ds(rb * LANES, LANES)] = v * SCALE
                return c
            lax.fori_loop(0, d_model, col_body, 0)

        for b in range(NBUF):
            gather_start(b, b)

        for b in range(NBUF):
            gather_wait(b, b)
            scale(b)
            scatter_start(b, b)
            gather_start(NBUF + b, NBUF + b)

        def pair_body(i, carry):
            grp0 = 1 + 2 * i
            for h in range(2):
                grp = grp0 + h
                p = (1 + h) % 2
                q = 1 - p
                for b in range(NBUF):
                    s = p * NBUF + b
                    sq = q * NBUF + b
                    g = grp * NBUF + b
                    gather_wait(g, s)
                    scale(s)
                    scatter_start(g, s)
                    scatter_wait((grp - 1) * NBUF + b, sq)
                    gather_start((grp + 1) * NBUF + b, sq)
            return carry

        lax.fori_loop(0, n_pairs, pair_body, 0)

        grp = n_groups - 1
        for b in range(NBUF):
            s = NBUF + b
            g = grp * NBUF + b
            gather_wait(g, s)
            scale(s)
            scatter_start(g, s)

        for b in range(NBUF):
            scatter_wait((n_groups - 2) * NBUF + b, b)
            scatter_wait((n_groups - 1) * NBUF + b, NBUF + b)

    return lookup


def kernel(x, table):
    n_rows, row_len = x.shape
    d_model = table.shape[1]
    x_t = jnp.swapaxes(x, 0, 1).astype(jnp.int32)
    out5 = _make_lookup(n_rows, row_len, d_model)(x_t, table)
    out = out5.transpose(2, 4, 0, 1, 3).reshape(n_rows, row_len, d_model)
    return out


# speedup vs baseline: 1.5718x; 1.5718x over previous
import functools
import math

import jax
import jax.numpy as jnp
from jax import lax
from jax.experimental import pallas as pl
from jax.experimental.pallas import tpu as pltpu
from jax.experimental.pallas import tpu_sc as plsc

D_MODEL = 64
SCALE = math.sqrt(D_MODEL)
LANES = 16

_info = plsc.get_sparse_core_info()
NUM_CORES = _info.num_cores
NUM_SUBCORES = _info.num_subcores
NUM_WORKERS = NUM_CORES * NUM_SUBCORES

CHUNK = 128
NSLOT = 8
INFLIGHT = 4
UNROLL = NSLOT


def _make_lookup(n_lookups, d_model):
    assert n_lookups % (NUM_WORKERS * CHUNK) == 0
    per_w = n_lookups // NUM_WORKERS
    n_chunks = per_w // CHUNK
    assert n_chunks % UNROLL == 0 and n_chunks // UNROLL >= 3
    n_oct = n_chunks // UNROLL
    assert d_model % LANES == 0

    mesh = plsc.VectorSubcoreMesh(core_axis_name="c", subcore_axis_name="s")

    @functools.partial(
        pl.kernel,
        mesh=mesh,
        compiler_params=pltpu.CompilerParams(use_tc_tiling_on_sc=False,
                                             needs_layout_passes=False),
        out_type=jax.ShapeDtypeStruct((n_lookups, d_model), jnp.float32),
        scratch_types=[
            pltpu.VMEM((n_chunks, CHUNK), jnp.int32),
            pltpu.VMEM((NSLOT, CHUNK, d_model), jnp.float32),
            pltpu.SemaphoreType.DMA((NSLOT,)),
            pltpu.SemaphoreType.DMA((NSLOT,)),
        ],
    )
    def lookup(x_hbm, table_hbm, out_hbm, idx_v, rows_v, gsem, ssem):
        wid = lax.axis_index("s") * NUM_CORES + lax.axis_index("c")
        base = wid * per_w

        pltpu.sync_copy(x_hbm.at[pl.ds(wid * n_chunks, n_chunks)], idx_v)

        def gather_start(g, s):
            pltpu.async_copy(table_hbm.at[idx_v.at[g]], rows_v.at[s],
                             gsem.at[s])

        def gather_wait(g, s):
            pltpu.make_async_copy(table_hbm.at[idx_v.at[g]], rows_v.at[s],
                                  gsem.at[s]).wait()

        def scatter_start(g, s):
            pltpu.async_copy(rows_v.at[s],
                             out_hbm.at[pl.ds(base + g * CHUNK, CHUNK)],
                             ssem.at[s])

        def scatter_wait(g, s):
            pltpu.make_async_copy(rows_v.at[s],
                                  out_hbm.at[pl.ds(base + g * CHUNK, CHUNK)],
                                  ssem.at[s]).wait()

        n_vec = d_model // LANES
        rows_per_it = 8

        def scale(s):
            def body(i, c):
                for k in range(rows_per_it):
                    r = i * rows_per_it + k
                    for l in range(n_vec):
                        v = rows_v[s, r, pl.ds(l * LANES, LANES)]
                        rows_v[s, r, pl.ds(l * LANES, LANES)] = v * SCALE
                return c
            lax.fori_loop(0, CHUNK // rows_per_it, body, 0)

        def step(g, j, first_oct, last_oct):
            gather_wait(g, j)
            scale(j)
            scatter_start(g, j)
            pre = g + INFLIGHT
            sp = (j + INFLIGHT) % NSLOT
            if last_oct and j >= UNROLL - INFLIGHT:
                return
            if not (first_oct and j < NSLOT - INFLIGHT):
                scatter_wait(pre - NSLOT, sp)
            gather_start(pre, sp)

        for b in range(INFLIGHT):
            gather_start(b, b)

        for j in range(UNROLL):
            step(j, j, True, False)

        def oct_body(o, carry):
            for j in range(UNROLL):
                step(o * UNROLL + j, j, False, False)
            return carry

        lax.fori_loop(1, n_oct - 1, oct_body, 0)

        g0 = (n_oct - 1) * UNROLL
        for j in range(UNROLL):
            step(g0 + j, j, False, True)
        for j in range(UNROLL):
            scatter_wait(g0 + j, j)

    return lookup


def kernel(x, table):
    n_rows, row_len = x.shape
    d_model = table.shape[1]
    n_lookups = n_rows * row_len
    xf = x.reshape(n_lookups // CHUNK, CHUNK).astype(jnp.int32)
    out2 = _make_lookup(n_lookups, d_model)(xf, table)
    return out2.reshape(n_rows, row_len, d_model)


# speedup vs baseline: 1.6886x; 1.0743x over previous
import functools
import math

import jax
import jax.numpy as jnp
from jax import lax
from jax.experimental import pallas as pl
from jax.experimental.pallas import tpu as pltpu
from jax.experimental.pallas import tpu_sc as plsc

D_MODEL = 64
SCALE = math.sqrt(D_MODEL)
LANES = 16

_info = plsc.get_sparse_core_info()
NUM_CORES = _info.num_cores
NUM_SUBCORES = _info.num_subcores
NUM_WORKERS = NUM_CORES * NUM_SUBCORES

RBLK = 128
SUBL = 8
NSLOT = 4
INFLIGHT = 2
UNROLL = 8


def _make_lookup(n_rows, row_len, d_model):
    assert n_rows == NUM_WORKERS * RBLK
    n_chunks = row_len
    n_jt = d_model // SUBL
    assert n_chunks % UNROLL == 0 and n_chunks // UNROLL >= 2
    n_ct = n_chunks // UNROLL
    assert d_model % LANES == 0

    mesh = plsc.VectorSubcoreMesh(core_axis_name="c", subcore_axis_name="s")

    @functools.partial(
        pl.kernel,
        mesh=mesh,
        compiler_params=pltpu.CompilerParams(use_tc_tiling_on_sc=False,
                                             needs_layout_passes=False),
        out_type=jax.ShapeDtypeStruct(
            (n_chunks, n_jt, NUM_WORKERS, SUBL, RBLK), jnp.float32),
        scratch_types=[
            pltpu.VMEM((n_ct, UNROLL, RBLK), jnp.int32),
            pltpu.VMEM((NSLOT, RBLK, d_model), jnp.float32),
            pltpu.VMEM((NSLOT, d_model, RBLK), jnp.float32),
            pltpu.SemaphoreType.DMA((NSLOT,)),
            pltpu.SemaphoreType.DMA((NSLOT,)),
        ],
    )
    def lookup(x_hbm, table_hbm, out_hbm, idx_v, rows_v, stage_v, gsem, ssem):
        wid = lax.axis_index("s") * NUM_CORES + lax.axis_index("c")

        pltpu.sync_copy(x_hbm.at[:, wid], idx_v)

        def gather_start(g, s):
            pltpu.async_copy(table_hbm.at[idx_v.at[g // UNROLL, g % UNROLL]],
                             rows_v.at[s], gsem.at[s])

        def gather_wait(g, s):
            pltpu.make_async_copy(
                table_hbm.at[idx_v.at[g // UNROLL, g % UNROLL]],
                rows_v.at[s], gsem.at[s]).wait()

        def scatter_start(g, s):
            for jt in range(n_jt):
                pltpu.async_copy(stage_v.at[s, pl.ds(jt * SUBL, SUBL)],
                                 out_hbm.at[g, jt, wid], ssem.at[s])

        def scatter_wait(g, s):
            for jt in range(n_jt):
                pltpu.make_async_copy(stage_v.at[s, pl.ds(jt * SUBL, SUBL)],
                                      out_hbm.at[g, jt, wid],
                                      ssem.at[s]).wait()

        iota = lax.iota(jnp.int32, LANES)
        rowidx = [iota + lg * LANES for lg in range(RBLK // LANES)]

        def scale(s):
            def diag_body(t, c):
                db = t // LANES
                k = t % LANES
                dv = ((iota + k) & (LANES - 1)) + db * LANES
                for lg in range(RBLK // LANES):
                    v = plsc.load_gather(rows_v.at[s], [rowidx[lg], dv])
                    plsc.store_scatter(stage_v.at[s], [dv, rowidx[lg]],
                                       v * SCALE)
                return c
            lax.fori_loop(0, d_model, diag_body, 0)

        def step(g, j, first_ct, last_ct):
            s = j % NSLOT
            gather_wait(g, s)
            scale(s)
            scatter_start(g, s)
            pre = g + INFLIGHT
            sp = (j + INFLIGHT) % NSLOT
            if last_ct and j >= UNROLL - INFLIGHT:
                return
            if not (first_ct and j < NSLOT - INFLIGHT):
                scatter_wait(pre - NSLOT, sp)
            gather_start(pre, sp)

        for b in range(INFLIGHT):
            gather_start(b, b)

        for j in range(UNROLL):
            step(j, j, True, False)

        def ct_body(o, carry):
            for j in range(UNROLL):
                step(o * UNROLL + j, j, False, False)
            return carry

        lax.fori_loop(1, n_ct - 1, ct_body, 0)

        g0 = (n_ct - 1) * UNROLL
        for j in range(UNROLL):
            step(g0 + j, j, False, True)
        for j in range(UNROLL - NSLOT, UNROLL):
            scatter_wait(g0 + j, j % NSLOT)

    return lookup


def kernel(x, table):
    n_rows, row_len = x.shape
    d_model = table.shape[1]
    xv = (x.astype(jnp.int32).T
          .reshape(row_len // SUBL, SUBL, n_rows // RBLK, RBLK)
          .transpose(0, 2, 1, 3))
    out5 = _make_lookup(n_rows, row_len, d_model)(xv, table)
    return out5.transpose(2, 4, 0, 1, 3).reshape(n_rows, row_len, d_model)
